# baseline (speedup 1.0000x reference)
import functools

import jax
import jax.numpy as jnp
from jax.experimental import pallas as pl

_MARGIN = 1.0
_BN = 512


def _triplet_kernel(a_ref, p_ref, out_ref):
    a = a_ref[...]
    p = p_ref[...]
    b, d = a.shape

    prods = jnp.concatenate([a * a, p * p, a * p], axis=1)
    grp = jax.lax.broadcasted_iota(jnp.int32, (3 * d, 3), 0) // d
    sel = (grp == jax.lax.broadcasted_iota(jnp.int32, (3 * d, 3), 1))
    sums = jnp.dot(prods, sel.astype(jnp.float32),
                   preferred_element_type=jnp.float32)
    an2 = sums[:, 0:1]
    pn2 = sums[:, 1:2]
    ap_dot = sums[:, 2:3]

    p_n = p * jax.lax.rsqrt(pn2)

    eye = (jax.lax.broadcasted_iota(jnp.int32, (_BN, _BN), 0)
           == jax.lax.broadcasted_iota(jnp.int32, (_BN, _BN), 1))

    dims = (((1,), (1,)), ((), ()))
    acc = jnp.full((b, 1), jnp.inf, jnp.float32)
    for j in range(b // _BN):
        lo, hi = j * _BN, (j + 1) * _BN
        tile = jax.lax.dot_general(a, p_n[lo:hi, :], dims,
                                   preferred_element_type=jnp.float32)
        m = jnp.min(tile, axis=1, keepdims=True)
        sub = jnp.where(eye, jnp.inf, tile[lo:hi, :])
        m_sub = jnp.min(sub, axis=1, keepdims=True)
        pieces = ([m[:lo]] if lo else []) + [m_sub] + ([m[hi:]] if hi < b else [])
        m = jnp.concatenate(pieces, axis=0) if len(pieces) > 1 else m_sub
        acc = jnp.minimum(acc, m)

    ra = jax.lax.rsqrt(an2)
    an_dist = acc * ra
    ap_dist = ap_dot / jnp.maximum(jnp.sqrt(an2) * jnp.sqrt(pn2), 1e-8)

    loss = jnp.sum(jax.nn.relu(_MARGIN + ap_dist - an_dist)) / b
    out_ref[...] = loss.reshape(1, 1)


@functools.partial(jax.jit, static_argnames=("interpret",))
def kernel(anchor, positive, interpret=False):
    out = pl.pallas_call(
        _triplet_kernel,
        out_shape=jax.ShapeDtypeStruct((1, 1), jnp.float32),
        interpret=interpret,
    )(anchor, positive)
    return out[0, 0]


# speedup vs baseline: 1.9528x; 1.9528x over previous
import functools

import jax
import jax.numpy as jnp
from jax.experimental import pallas as pl

_MARGIN = 1.0
_BN = 512


def _triplet_kernel(a_ref, p_ref, out_ref):
    a = a_ref[...]
    p = p_ref[...]
    b, _ = a.shape

    an2 = jnp.sum(a * a, axis=1, keepdims=True)
    pn2 = jnp.sum(p * p, axis=1, keepdims=True)
    ap_dot = jnp.sum(a * p, axis=1, keepdims=True)
    p_n = p * jax.lax.rsqrt(pn2)

    eye = (jax.lax.broadcasted_iota(jnp.int32, (_BN, _BN), 0)
           == jax.lax.broadcasted_iota(jnp.int32, (_BN, _BN), 1))

    acc = jnp.full((b, 1), jnp.inf, jnp.float32)
    for j in range(b // _BN):
        lo, hi = j * _BN, (j + 1) * _BN
        tile = jnp.dot(a, p_n[lo:hi, :].T,
                       preferred_element_type=jnp.float32)
        m = jnp.min(tile, axis=1, keepdims=True)
        sub = jnp.where(eye, jnp.inf, tile[lo:hi, :])
        m_sub = jnp.min(sub, axis=1, keepdims=True)
        pieces = ([m[:lo]] if lo else []) + [m_sub] + ([m[hi:]] if hi < b else [])
        m = jnp.concatenate(pieces, axis=0) if len(pieces) > 1 else m_sub
        acc = jnp.minimum(acc, m)

    an_dist = acc * jax.lax.rsqrt(an2)
    ap_dist = ap_dot / jnp.maximum(jnp.sqrt(an2) * jnp.sqrt(pn2), 1e-8)

    loss = jnp.sum(jax.nn.relu(_MARGIN + ap_dist - an_dist)) / b
    out_ref[...] = loss.reshape(1, 1)


@functools.partial(jax.jit, static_argnames=("interpret",))
def kernel(anchor, positive, interpret=False):
    out = pl.pallas_call(
        _triplet_kernel,
        out_shape=jax.ShapeDtypeStruct((1, 1), jnp.float32),
        interpret=interpret,
    )(anchor, positive)
    return out[0, 0]


# speedup vs baseline: 3.2086x; 1.6431x over previous
import functools

import jax
import jax.numpy as jnp
from jax.experimental import pallas as pl

_MARGIN = 1.0
_BN = 512


def _triplet_kernel(a_ref, p_ref, out_ref):
    a_t = a_ref[...]
    p_t = p_ref[...]
    _, b = a_t.shape

    an2 = jnp.sum(a_t * a_t, axis=0, keepdims=True)
    pn2 = jnp.sum(p_t * p_t, axis=0, keepdims=True)
    ap_dot = jnp.sum(a_t * p_t, axis=0, keepdims=True)
    p_n = (p_t * jax.lax.rsqrt(pn2)).astype(jnp.bfloat16)
    a_h = a_t.astype(jnp.bfloat16)

    eye = (jax.lax.broadcasted_iota(jnp.int32, (_BN, _BN), 0)
           == jax.lax.broadcasted_iota(jnp.int32, (_BN, _BN), 1))
    dims = (((0,), (0,)), ((), ()))

    acc = jnp.full((1, b), jnp.inf, jnp.float32)
    for j in range(b // _BN):
        lo, hi = j * _BN, (j + 1) * _BN
        tile = jax.lax.dot_general(p_n[:, lo:hi], a_h, dims,
                                   preferred_element_type=jnp.float32)
        m = jnp.min(tile, axis=0, keepdims=True)
        sub = jnp.where(eye, jnp.inf, tile[:, lo:hi])
        m_sub = jnp.min(sub, axis=0, keepdims=True)
        pieces = (([m[:, :lo]] if lo else []) + [m_sub]
                  + ([m[:, hi:]] if hi < b else []))
        m = jnp.concatenate(pieces, axis=1) if len(pieces) > 1 else m_sub
        acc = jnp.minimum(acc, m)

    an_dist = acc * jax.lax.rsqrt(an2)
    ap_dist = ap_dot / jnp.maximum(jnp.sqrt(an2) * jnp.sqrt(pn2), 1e-8)

    loss = jnp.sum(jax.nn.relu(_MARGIN + ap_dist - an_dist)) / b
    out_ref[...] = loss.reshape(1, 1)


@functools.partial(jax.jit, static_argnames=("interpret",))
def kernel(anchor, positive, interpret=False):
    out = pl.pallas_call(
        _triplet_kernel,
        out_shape=jax.ShapeDtypeStruct((1, 1), jnp.float32),
        interpret=interpret,
    )(anchor.T, positive.T)
    return out[0, 0]
